# Initial kernel scaffold; baseline (speedup 1.0000x reference)
import jax
import jax.numpy as jnp
from jax.experimental import pallas as pl


def kernel(x, edge_index, W1, b1, W2, b2):
    raise NotImplementedError("write your pallas kernel here")


# speedup vs baseline: 21.2858x; 21.2858x over previous
import functools

import jax
import jax.numpy as jnp
from jax import lax
from jax.experimental import pallas as pl
from jax.experimental.pallas import tpu as pltpu
from jax.experimental.pallas import tpu_sc as plsc

N_NODES = 10000
NP = 10240
NSUB = 16
RPT = NP // NSUB
K = 128
T = 158
EPAD = NSUB * T * K


def _make_agg(df):
  mesh = plsc.VectorSubcoreMesh(core_axis_name="c", subcore_axis_name="s")

  @functools.partial(
      pl.kernel,
      out_type=jax.ShapeDtypeStruct((2 * NP, df), jnp.float32),
      mesh=mesh,
      compiler_params=pltpu.CompilerParams(use_tc_tiling_on_sc=False),
      scratch_types=[
          pltpu.VMEM((T, K), jnp.int32),
          pltpu.VMEM((T, K), jnp.int32),
          pltpu.VMEM((2, K, df), jnp.float32),
          pltpu.VMEM_SHARED((NP, df), jnp.float32),
          pltpu.SemaphoreType.DMA((2,)),
      ],
  )
  def agg(tab, src32, dst16, out, src_v, dst_v, rows, acc, sems):
    c = lax.axis_index("c")
    s = lax.axis_index("s")
    w = c * NSUB + s
    r0 = s * RPT
    pltpu.sync_copy(tab.at[pl.ds(c * NP + r0, RPT)], acc.at[pl.ds(r0, RPT)])
    pltpu.sync_copy(src32.at[w], src_v)
    pltpu.sync_copy(dst16.at[s], dst_v)
    plsc.subcore_barrier()

    def start(j, b):
      return pltpu.async_copy(tab.at[src_v.at[j]], rows.at[b], sems.at[b])

    def wait(j, b):
      pltpu.make_async_copy(tab.at[src_v.at[j]], rows.at[b], sems.at[b]).wait()

    def scat(j, b):
      pltpu.sync_copy(rows.at[b], acc.at[dst_v.at[j]], add=True)

    start(0, 0)

    def body(i, carry):
      j = 2 * i
      start(j + 1, 1)
      wait(j, 0)
      scat(j, 0)

      @pl.when(j + 2 < T)
      def _():
        start(j + 2, 0)

      wait(j + 1, 1)
      scat(j + 1, 1)
      return carry

    lax.fori_loop(0, T // 2, body, 0)
    plsc.subcore_barrier()
    pltpu.sync_copy(acc.at[pl.ds(r0, RPT)], out.at[pl.ds(c * NP + r0, RPT)])

  return agg


def _make_deg():
  mesh = plsc.VectorSubcoreMesh(core_axis_name="c", subcore_axis_name="s")

  @functools.partial(
      pl.kernel,
      out_type=jax.ShapeDtypeStruct((2 * NP, 8), jnp.float32),
      mesh=mesh,
      compiler_params=pltpu.CompilerParams(use_tc_tiling_on_sc=False),
      scratch_types=[
          pltpu.VMEM((T, K), jnp.int32),
          pltpu.VMEM((K, 8), jnp.float32),
          pltpu.VMEM_SHARED((NP, 8), jnp.float32),
      ],
  )
  def deg(dst16, ones_h, zer_h, out, dst_v, ones_v, acc):
    c = lax.axis_index("c")
    s = lax.axis_index("s")
    r0 = s * RPT
    pltpu.sync_copy(zer_h.at[pl.ds(r0, RPT)], acc.at[pl.ds(r0, RPT)])
    pltpu.sync_copy(dst16.at[s], dst_v)
    pltpu.sync_copy(ones_h, ones_v)
    plsc.subcore_barrier()
    lo = c * (T // 2)

    def body(j, carry):
      pltpu.sync_copy(ones_v, acc.at[dst_v.at[j]], add=True)
      return carry

    lax.fori_loop(lo, lo + T // 2, body, 0)
    plsc.subcore_barrier()
    pltpu.sync_copy(acc.at[pl.ds(r0, RPT)], out.at[pl.ds(c * NP + r0, RPT)])

  return deg


_agg64 = _make_agg(64)
_agg32 = _make_agg(32)
_deg = _make_deg()

RB = 1280


def _pre_body(d_ref, x_ref, dinv_ref, xs_ref):
  deg = d_ref[0, :, 0:1] + d_ref[1, :, 0:1] + 1.0
  dinv = lax.rsqrt(deg)
  dinv_ref[...] = dinv
  xs_ref[0] = x_ref[:, :64] * dinv
  xs_ref[1] = x_ref[:, 64:] * dinv


def _mid_body(a_ref, dinv_ref, w1_ref, b1_ref, w2_ref, gs_ref):
  dv = dinv_ref[...]
  y0 = a_ref[0] * dv
  y1 = a_ref[1] * dv
  h = (jnp.dot(y0, w1_ref[:64, :], preferred_element_type=jnp.float32)
       + jnp.dot(y1, w1_ref[64:, :], preferred_element_type=jnp.float32)
       + b1_ref[...])
  h = jnp.maximum(h, 0.0)
  g = jnp.dot(h, w2_ref[...], preferred_element_type=jnp.float32) * dv
  gs_ref[0] = g[:, :32]
  gs_ref[1] = g[:, 32:]


def _fin_body(a_ref, dinv_ref, b2_ref, o_ref):
  dv = dinv_ref[...]
  l0 = a_ref[0] * dv + b2_ref[:, :32]
  l1 = a_ref[1] * dv + b2_ref[:, 32:]
  m = jnp.maximum(jnp.max(l0, axis=-1, keepdims=True),
                  jnp.max(l1, axis=-1, keepdims=True))
  lse = m + jnp.log(jnp.sum(jnp.exp(l0 - m), axis=-1, keepdims=True)
                    + jnp.sum(jnp.exp(l1 - m), axis=-1, keepdims=True))
  o_ref[:, :32] = l0 - lse
  o_ref[:, 32:] = l1 - lse


def kernel(x, edge_index, W1, b1, W2, b2):
  n = x.shape[0]
  e = edge_index.shape[1]
  src = edge_index[0]
  dst = edge_index[1]
  pad = EPAD - e
  src_p = jnp.concatenate([src, jnp.zeros((pad,), jnp.int32)])
  dst_p = jnp.concatenate([dst, jnp.full((pad,), n, jnp.int32)])
  src16 = src_p.reshape(NSUB, T, K)
  dst16 = dst_p.reshape(NSUB, T, K)
  src32 = jnp.concatenate([src16, src16 + NP], axis=0)

  ones_h = jnp.zeros((K, 8), jnp.float32).at[:, 0].set(1.0)
  zer_h = jnp.zeros((NP, 8), jnp.float32)
  deg8 = _deg(dst16, ones_h, zer_h).reshape(2, NP, 8)

  x_p = jnp.pad(x, ((0, NP - n), (0, 0)))
  nblk = NP // RB
  dinv, xs = pl.pallas_call(
      _pre_body,
      grid=(nblk,),
      in_specs=[
          pl.BlockSpec((2, RB, 8), lambda i: (0, i, 0)),
          pl.BlockSpec((RB, 128), lambda i: (i, 0)),
      ],
      out_specs=[
          pl.BlockSpec((RB, 1), lambda i: (i, 0)),
          pl.BlockSpec((2, RB, 64), lambda i: (0, i, 0)),
      ],
      out_shape=[
          jax.ShapeDtypeStruct((NP, 1), jnp.float32),
          jax.ShapeDtypeStruct((2, NP, 64), jnp.float32),
      ],
  )(deg8, x_p)

  agg1 = _agg64(xs.reshape(2 * NP, 64), src32, dst16).reshape(2, NP, 64)

  gs = pl.pallas_call(
      _mid_body,
      grid=(nblk,),
      in_specs=[
          pl.BlockSpec((2, RB, 64), lambda i: (0, i, 0)),
          pl.BlockSpec((RB, 1), lambda i: (i, 0)),
          pl.BlockSpec((128, 128), lambda i: (0, 0)),
          pl.BlockSpec((1, 128), lambda i: (0, 0)),
          pl.BlockSpec((128, 64), lambda i: (0, 0)),
      ],
      out_specs=pl.BlockSpec((2, RB, 32), lambda i: (0, i, 0)),
      out_shape=jax.ShapeDtypeStruct((2, NP, 32), jnp.float32),
  )(agg1, dinv, W1, b1.reshape(1, -1), W2)

  agg2 = _agg32(gs.reshape(2 * NP, 32), src32, dst16).reshape(2, NP, 32)

  out = pl.pallas_call(
      _fin_body,
      grid=(nblk,),
      in_specs=[
          pl.BlockSpec((2, RB, 32), lambda i: (0, i, 0)),
          pl.BlockSpec((RB, 1), lambda i: (i, 0)),
          pl.BlockSpec((1, 64), lambda i: (0, 0)),
      ],
      out_specs=pl.BlockSpec((RB, 64), lambda i: (i, 0)),
      out_shape=jax.ShapeDtypeStruct((n, W2.shape[1]), jnp.float32),
  )(agg2, dinv, b2.reshape(1, -1))
  return out
